# baseline (speedup 1.0000x reference)
import functools

import jax
import jax.numpy as jnp
from jax import lax
from jax.experimental import pallas as pl
from jax.experimental.pallas import tpu as pltpu
from jax.experimental.pallas import tpu_sc as plsc

_C = 16

_GDN = lax.GatherDimensionNumbers(
    offset_dims=(), collapsed_slice_dims=(0,), start_index_map=(0,)
)


@functools.cache
def _sc_call(N, D, V):
    info = plsc.get_sparse_core_info()
    nw = info.num_cores * info.num_subcores
    rows_w = N // nw
    n_chunks = rows_w // _C
    assert n_chunks % 2 == 0

    mesh = plsc.VectorSubcoreMesh(core_axis_name="c", subcore_axis_name="s")

    @functools.partial(
        pl.kernel,
        mesh=mesh,
        compiler_params=pltpu.CompilerParams(needs_layout_passes=False),
        out_type=jax.ShapeDtypeStruct((N, D), jnp.float32),
        scratch_types=[
            pltpu.VMEM((V * D,), jnp.float32),
            pltpu.VMEM((rows_w,), jnp.int32),
            pltpu.VMEM((2, _C, D), jnp.float32),
            pltpu.SemaphoreType.DMA,
            pltpu.SemaphoreType.DMA,
            pltpu.SemaphoreType.DMA,
            pltpu.SemaphoreType.DMA,
        ],
    )
    def k(x_hbm, idx_hbm, table_hbm, out_hbm, table_v, idx_v, x_bufs,
          inx0, inx1, outs0, outs1):
        in_sems = (inx0, inx1)
        out_sems = (outs0, outs1)
        wid = lax.axis_index("s") * info.num_cores + lax.axis_index("c")
        base = wid * rows_w
        pltpu.sync_copy(idx_hbm.at[pl.ds(base, rows_w)], idx_v)
        pltpu.sync_copy(table_hbm, table_v)

        def issue_in(ci, b):
            r0 = base + ci * _C
            pltpu.async_copy(x_hbm.at[pl.ds(r0, _C)], x_bufs.at[b], in_sems[b])

        def wait_in(b):
            pltpu.make_async_copy(
                x_hbm.at[pl.ds(base, _C)], x_bufs.at[b], in_sems[b]
            ).wait()

        def wait_out(b):
            pltpu.make_async_copy(
                x_bufs.at[b], out_hbm.at[pl.ds(base, _C)], out_sems[b]
            ).wait()

        cols = [lax.iota(jnp.int32, 16) + c * 16 for c in range(D // 16)]

        def compute(ci, b):
            def row_body(r, rcarry):
                bidx = jnp.full((16,), ci * _C + r, jnp.int32)
                brow = plsc.load_gather(idx_v, [bidx]) * D
                for c in range(D // 16):
                    e = plsc.load_gather(table_v, [brow + cols[c]])
                    sl = pl.ds(c * 16, 16)
                    x_bufs[b, r, sl] = x_bufs[b, r, sl] + e
                return rcarry

            lax.fori_loop(0, _C, row_body, 0, unroll=False)

        issue_in(0, 0)

        def step(ci, b, b2):
            @pl.when(ci + 1 < n_chunks)
            def _():
                @pl.when(ci >= 1)
                def _():
                    wait_out(b2)

                issue_in(ci + 1, b2)

            wait_in(b)
            compute(ci, b)
            r0 = base + ci * _C
            pltpu.async_copy(x_bufs.at[b], out_hbm.at[pl.ds(r0, _C)], out_sems[b])

        def group_body(g, carry):
            step(2 * g, 0, 1)
            step(2 * g + 1, 1, 0)
            return carry

        lax.fori_loop(0, n_chunks // 2, group_body, 0, unroll=False)
        wait_out(0)
        wait_out(1)

    return k


def kernel(x, indices, table):
    B, S, D = x.shape
    V = table.shape[0]
    N = B * S
    x2 = x.reshape(N, D)
    idx2 = indices.reshape(N).astype(jnp.int32)
    out = _sc_call(N, D, V)(x2, idx2, table.reshape(V * D))
    return out.reshape(B, S, D)


# speedup vs baseline: 1.6697x; 1.6697x over previous
import functools

import jax
import jax.numpy as jnp
from jax import lax
from jax.experimental import pallas as pl
from jax.experimental.pallas import tpu as pltpu
from jax.experimental.pallas import tpu_sc as plsc

_C = 8
_NBUF = 4


@functools.cache
def _sc_call(N, D, V):
    info = plsc.get_sparse_core_info()
    nw = info.num_cores * info.num_subcores
    rows_w = N // nw
    n_chunks = rows_w // _C
    assert n_chunks % _NBUF == 0 and n_chunks >= 2 * _NBUF

    mesh = plsc.VectorSubcoreMesh(core_axis_name="c", subcore_axis_name="s")

    @functools.partial(
        pl.kernel,
        mesh=mesh,
        compiler_params=pltpu.CompilerParams(needs_layout_passes=False),
        out_type=jax.ShapeDtypeStruct((N, D), jnp.float32),
        scratch_types=[
            pltpu.VMEM((rows_w,), jnp.int32),
            pltpu.VMEM((_NBUF, _C, D), jnp.float32),
            pltpu.VMEM((_NBUF, _C, D), jnp.float32),
        ]
        + [pltpu.SemaphoreType.DMA] * (3 * _NBUF),
    )
    def k(x_hbm, idx_hbm, table_hbm, out_hbm, idx_v, x_bufs, emb_bufs, *sems):
        in_x_sems = sems[0:_NBUF]
        in_e_sems = sems[_NBUF:2 * _NBUF]
        out_sems = sems[2 * _NBUF:3 * _NBUF]
        wid = lax.axis_index("s") * info.num_cores + lax.axis_index("c")
        base = wid * rows_w
        pltpu.sync_copy(idx_hbm.at[pl.ds(base, rows_w)], idx_v)

        def issue_in(ci, b):
            r0 = base + ci * _C
            pltpu.async_copy(x_hbm.at[pl.ds(r0, _C)], x_bufs.at[b], in_x_sems[b])
            pltpu.async_copy(
                table_hbm.at[idx_v.at[pl.ds(ci * _C, _C)]],
                emb_bufs.at[b],
                in_e_sems[b],
            )

        def wait_in(ci, b):
            pltpu.make_async_copy(
                x_hbm.at[pl.ds(base, _C)], x_bufs.at[b], in_x_sems[b]
            ).wait()
            pltpu.make_async_copy(
                table_hbm.at[idx_v.at[pl.ds(ci * _C, _C)]],
                emb_bufs.at[b],
                in_e_sems[b],
            ).wait()

        def wait_out(b):
            pltpu.make_async_copy(
                x_bufs.at[b], out_hbm.at[pl.ds(base, _C)], out_sems[b]
            ).wait()

        def compute(b):
            def row_body(r, rcarry):
                for c in range(D // 16):
                    sl = pl.ds(c * 16, 16)
                    x_bufs[b, r, sl] = x_bufs[b, r, sl] + emb_bufs[b, r, sl]
                return rcarry

            lax.fori_loop(0, _C, row_body, 0, unroll=False)

        for b in range(_NBUF - 1):
            issue_in(b, b)

        def step(ci, b):
            b_next = (b + _NBUF - 1) % _NBUF

            @pl.when(ci + _NBUF - 1 < n_chunks)
            def _():
                @pl.when(ci >= 1)
                def _():
                    wait_out(b_next)

                issue_in(ci + _NBUF - 1, b_next)

            wait_in(ci, b)
            compute(b)
            r0 = base + ci * _C
            pltpu.async_copy(x_bufs.at[b], out_hbm.at[pl.ds(r0, _C)], out_sems[b])

        def group_body(g, carry):
            for b in range(_NBUF):
                step(_NBUF * g + b, b)
            return carry

        lax.fori_loop(0, n_chunks // _NBUF, group_body, 0, unroll=False)
        for b in range(_NBUF):
            wait_out(b)

    return k


def kernel(x, indices, table):
    B, S, D = x.shape
    V = table.shape[0]
    N = B * S
    x2 = x.reshape(N, D)
    idx2 = indices.reshape(N).astype(jnp.int32)
    out = _sc_call(N, D, V)(x2, idx2, table)
    return out.reshape(B, S, D)
